# Initial kernel scaffold; baseline (speedup 1.0000x reference)
import jax
import jax.numpy as jnp
from jax.experimental import pallas as pl


def kernel(xyz_1, xyz_2, points_1, points_2, W1, b1, W2, b2):
    raise NotImplementedError("write your pallas kernel here")


# speedup vs baseline: 33.4250x; 33.4250x over previous
import functools

import jax
import jax.numpy as jnp
from jax.experimental import pallas as pl
from jax.experimental.pallas import tpu as pltpu


def _p1_kernel(points_1_ref, w1_ref, b1_ref, out_ref):
    out_ref[0] = (
        jnp.dot(w1_ref[...], points_1_ref[0], preferred_element_type=jnp.float32)
        + b1_ref[...]
    )


def _interp_kernel(xyz1_ref, xyz2_ref, p2_ref, p1_ref, w2_ref, b2_ref, out_ref,
                   *, M, TN, K):
    x1 = xyz1_ref[0]
    x2 = xyz2_ref[0]

    d0 = x1[0][:, None] - x2[0][None, :]
    d1 = x1[1][:, None] - x2[1][None, :]
    d2 = x1[2][:, None] - x2[2][None, :]
    D = d0 * d0 + d1 * d1 + d2 * d2

    iota0 = jax.lax.broadcasted_iota(jnp.int32, (M, TN), 0)

    recips = []
    masks = []
    for _ in range(K):
        dmin = jnp.min(D, axis=0, keepdims=True)
        imin = jnp.min(jnp.where(D == dmin, iota0, M), axis=0, keepdims=True)
        sel = iota0 == imin
        recips.append(1.0 / (dmin + 0.1))
        masks.append(sel)
        D = jnp.where(sel, jnp.inf, D)

    norm = recips[0] + recips[1] + recips[2]
    St = jnp.zeros((M, TN), jnp.float32)
    for r, sel in zip(recips, masks):
        St = jnp.where(sel, r / norm, St)

    interp = jnp.dot(p1_ref[0], St, preferred_element_type=jnp.float32)
    p2 = (
        jnp.dot(w2_ref[...], p2_ref[0], preferred_element_type=jnp.float32)
        + b2_ref[...]
    )
    out_ref[0] = interp + p2


def kernel(xyz_1, xyz_2, points_1, points_2, W1, b1, W2, b2):
    B, _, M = xyz_1.shape
    N = xyz_2.shape[2]
    C = points_1.shape[1]
    O = W1.shape[0]
    K = 3
    TN = 256
    NB = N // TN

    b1c = b1.reshape(O, 1)
    b2c = b2.reshape(O, 1)

    p1 = pl.pallas_call(
        _p1_kernel,
        grid=(B,),
        in_specs=[
            pl.BlockSpec((1, C, M), lambda b: (b, 0, 0)),
            pl.BlockSpec((O, C), lambda b: (0, 0)),
            pl.BlockSpec((O, 1), lambda b: (0, 0)),
        ],
        out_specs=pl.BlockSpec((1, O, M), lambda b: (b, 0, 0)),
        out_shape=jax.ShapeDtypeStruct((B, O, M), jnp.float32),
    )(points_1, W1, b1c)

    out = pl.pallas_call(
        functools.partial(_interp_kernel, M=M, TN=TN, K=K),
        grid=(B, NB),
        in_specs=[
            pl.BlockSpec((1, 3, M), lambda b, nb: (b, 0, 0)),
            pl.BlockSpec((1, 3, TN), lambda b, nb: (b, 0, nb)),
            pl.BlockSpec((1, O, TN), lambda b, nb: (b, 0, nb)),
            pl.BlockSpec((1, O, M), lambda b, nb: (b, 0, 0)),
            pl.BlockSpec((O, O), lambda b, nb: (0, 0)),
            pl.BlockSpec((O, 1), lambda b, nb: (0, 0)),
        ],
        out_specs=pl.BlockSpec((1, O, TN), lambda b, nb: (b, 0, nb)),
        out_shape=jax.ShapeDtypeStruct((B, O, N), jnp.float32),
    )(xyz_1, xyz_2, points_2, p1, W2, b2c)

    return (xyz_2, out)


# speedup vs baseline: 46.9512x; 1.4047x over previous
import functools

import jax
import jax.numpy as jnp
from jax.experimental import pallas as pl
from jax.experimental.pallas import tpu as pltpu


def _p1_kernel(points_1_ref, w1_ref, b1_ref, out_ref):
    out_ref[0] = (
        jnp.dot(w1_ref[...], points_1_ref[0], preferred_element_type=jnp.float32)
        + b1_ref[...]
    )


def _interp_kernel(xyz1_ref, xyz2_ref, p2_ref, p1_ref, w2_ref, b2_ref, out_ref,
                   *, M, TN, K):
    x1 = xyz1_ref[0]
    x2 = xyz2_ref[0]

    d0 = x1[0][:, None] - x2[0][None, :]
    d1 = x1[1][:, None] - x2[1][None, :]
    d2 = x1[2][:, None] - x2[2][None, :]
    D = d0 * d0 + d1 * d1 + d2 * d2

    m0 = jnp.min(D, axis=0, keepdims=True)
    D1 = jnp.where(D == m0, jnp.inf, D)
    m1 = jnp.min(D1, axis=0, keepdims=True)
    D2 = jnp.where(D1 == m1, jnp.inf, D1)
    m2 = jnp.min(D2, axis=0, keepdims=True)

    r0 = 1.0 / (m0 + 0.1)
    r1 = 1.0 / (m1 + 0.1)
    r2 = 1.0 / (m2 + 0.1)
    norm = r0 + r1 + r2
    w0 = r0 / norm
    w1 = r1 / norm
    w2 = r2 / norm

    St = jnp.where(
        D == m0, w0, jnp.where(D == m1, w1, jnp.where(D == m2, w2, 0.0))
    )

    interp = jnp.dot(p1_ref[0], St, preferred_element_type=jnp.float32)
    p2 = (
        jnp.dot(w2_ref[...], p2_ref[0], preferred_element_type=jnp.float32)
        + b2_ref[...]
    )
    out_ref[0] = interp + p2


def kernel(xyz_1, xyz_2, points_1, points_2, W1, b1, W2, b2):
    B, _, M = xyz_1.shape
    N = xyz_2.shape[2]
    C = points_1.shape[1]
    O = W1.shape[0]
    K = 3
    TN = 256
    NB = N // TN

    b1c = b1.reshape(O, 1)
    b2c = b2.reshape(O, 1)

    p1 = pl.pallas_call(
        _p1_kernel,
        grid=(B,),
        in_specs=[
            pl.BlockSpec((1, C, M), lambda b: (b, 0, 0)),
            pl.BlockSpec((O, C), lambda b: (0, 0)),
            pl.BlockSpec((O, 1), lambda b: (0, 0)),
        ],
        out_specs=pl.BlockSpec((1, O, M), lambda b: (b, 0, 0)),
        out_shape=jax.ShapeDtypeStruct((B, O, M), jnp.float32),
    )(points_1, W1, b1c)

    out = pl.pallas_call(
        functools.partial(_interp_kernel, M=M, TN=TN, K=K),
        grid=(B, NB),
        in_specs=[
            pl.BlockSpec((1, 3, M), lambda b, nb: (b, 0, 0)),
            pl.BlockSpec((1, 3, TN), lambda b, nb: (b, 0, nb)),
            pl.BlockSpec((1, O, TN), lambda b, nb: (b, 0, nb)),
            pl.BlockSpec((1, O, M), lambda b, nb: (b, 0, 0)),
            pl.BlockSpec((O, O), lambda b, nb: (0, 0)),
            pl.BlockSpec((O, 1), lambda b, nb: (0, 0)),
        ],
        out_specs=pl.BlockSpec((1, O, TN), lambda b, nb: (b, 0, nb)),
        out_shape=jax.ShapeDtypeStruct((B, O, N), jnp.float32),
    )(xyz_1, xyz_2, points_2, p1, W2, b2c)

    return (xyz_2, out)
